# baseline (speedup 1.0000x reference)
import functools

import jax
import jax.numpy as jnp
from jax import lax
from jax.experimental import pallas as pl
from jax.experimental.pallas import tpu as pltpu
from jax.experimental.pallas import tpu_sc as plsc

SEQ = 200
BATCH = 4096
VOCAB = 100000
NC, NS, L = 2, 16, 16
NW = NC * NS
COLS = BATCH // NW
CGRP = COLS // L
PAD_VOCAB = 100352
CHUNK = PAD_VOCAB // NS
RCHUNK = 40
NCHUNK = SEQ // RCHUNK


def _sc_kernel():
  mesh = plsc.VectorSubcoreMesh(core_axis_name="c", subcore_axis_name="s")

  @functools.partial(
      pl.kernel,
      out_type=jax.ShapeDtypeStruct((BATCH,), jnp.float32),
      mesh=mesh,
      compiler_params=pltpu.CompilerParams(needs_layout_passes=False),
      scratch_types=[
          pltpu.VMEM((PAD_VOCAB,), jnp.float32),
          pltpu.VMEM_SHARED((PAD_VOCAB,), jnp.float32),
          pltpu.VMEM((RCHUNK, COLS), jnp.int32),
          pltpu.VMEM((RCHUNK, COLS), jnp.int32),
          pltpu.VMEM((COLS,), jnp.float32),
          pltpu.VMEM((L,), jnp.float32),
          pltpu.SemaphoreType.DMA,
          pltpu.SemaphoreType.DMA,
          pltpu.SemaphoreType.DMA,
          pltpu.SemaphoreType.DMA,
      ],
  )
  def k(text_hbm, w_hbm, b_hbm, out_hbm, table_v, table_sh, idx0, idx1, out_v,
        b_v, sem_w, sem_t, sem_i0, sem_i1):
    sid = lax.axis_index("s")
    wid = sid * NC + lax.axis_index("c")
    base = wid * COLS
    bufs = [idx0, idx1]
    sems = [sem_i0, sem_i1]

    off = sid * CHUNK
    cp_stage = pltpu.async_copy(w_hbm.at[pl.ds(off, CHUNK)],
                                table_sh.at[pl.ds(off, CHUNK)], sem_w)
    cps = {}
    for c in range(2):
      cps[c] = pltpu.async_copy(
          text_hbm.at[pl.ds(c * RCHUNK, RCHUNK), pl.ds(base, COLS)],
          bufs[c], sems[c])
    pltpu.sync_copy(b_hbm, b_v)
    cp_stage.wait()
    plsc.subcore_barrier()
    pltpu.sync_copy(table_sh, table_v)

    bias = b_v[...]
    zero = jnp.zeros((L,), jnp.float32)
    accs = (zero,) * CGRP
    for c in range(NCHUNK):
      buf = bufs[c % 2]
      cps[c].wait()

      def row(i, a, buf=buf):
        return tuple(
            a[g] + plsc.load_gather(table_v, [buf[i, pl.ds(g * L, L)]])
            for g in range(CGRP)
        )

      accs = lax.fori_loop(0, RCHUNK, row, accs)
      if c + 2 < NCHUNK:
        cps[c + 2] = pltpu.async_copy(
            text_hbm.at[pl.ds((c + 2) * RCHUNK, RCHUNK), pl.ds(base, COLS)],
            bufs[c % 2], sems[c % 2])

    for g in range(CGRP):
      out_v[pl.ds(g * L, L)] = accs[g] + bias
    pltpu.sync_copy(out_v, out_hbm.at[pl.ds(base, COLS)])

  return k


def kernel(text, w, b):
  w_flat = jnp.pad(w.reshape(VOCAB), (0, PAD_VOCAB - VOCAB))
  b16 = jnp.broadcast_to(b, (L,)).astype(jnp.float32)
  return _sc_kernel()(text, w_flat, b16)


# speedup vs baseline: 1.0182x; 1.0182x over previous
import functools

import jax
import jax.numpy as jnp
from jax import lax
from jax.experimental import pallas as pl
from jax.experimental.pallas import tpu as pltpu
from jax.experimental.pallas import tpu_sc as plsc

SEQ = 200
BATCH = 4096
VOCAB = 100000
NC, NS, L = 2, 16, 16
NW = NC * NS
COLS = BATCH // NW
CGRP = COLS // L
PAD_VOCAB = 100352
CHUNK = PAD_VOCAB // NS
RCHUNK = 40
NCHUNK = SEQ // RCHUNK


def _sc_kernel():
  mesh = plsc.VectorSubcoreMesh(core_axis_name="c", subcore_axis_name="s")

  @functools.partial(
      pl.kernel,
      out_type=jax.ShapeDtypeStruct((BATCH,), jnp.float32),
      mesh=mesh,
      compiler_params=pltpu.CompilerParams(needs_layout_passes=False),
      scratch_types=[
          pltpu.VMEM((PAD_VOCAB,), jnp.float32),
          pltpu.VMEM_SHARED((PAD_VOCAB,), jnp.float32),
          pltpu.VMEM((RCHUNK, COLS), jnp.int32),
          pltpu.VMEM((RCHUNK, COLS), jnp.int32),
          pltpu.VMEM((RCHUNK, COLS), jnp.int32),
          pltpu.VMEM((RCHUNK, COLS), jnp.int32),
          pltpu.VMEM((COLS,), jnp.float32),
          pltpu.VMEM((L,), jnp.float32),
          pltpu.SemaphoreType.DMA,
          pltpu.SemaphoreType.DMA,
          pltpu.SemaphoreType.DMA,
          pltpu.SemaphoreType.DMA,
          pltpu.SemaphoreType.DMA,
      ],
  )
  def k(text_hbm, w_hbm, b_hbm, out_hbm, table_v, table_sh, idx0, idx1, idx2,
        idx3, out_v, b_v, sem_w, sem_i0, sem_i1, sem_i2, sem_i3):
    sid = lax.axis_index("s")
    wid = sid * NC + lax.axis_index("c")
    base = wid * COLS
    bufs = [idx0, idx1, idx2, idx3]
    sems = [sem_i0, sem_i1, sem_i2, sem_i3]
    NB = len(bufs)

    off = sid * CHUNK
    cp_stage = pltpu.async_copy(w_hbm.at[pl.ds(off, CHUNK)],
                                table_sh.at[pl.ds(off, CHUNK)], sem_w)
    cps = {}
    for c in range(NB):
      cps[c] = pltpu.async_copy(
          text_hbm.at[pl.ds(c * RCHUNK, RCHUNK), pl.ds(base, COLS)],
          bufs[c], sems[c])
    pltpu.sync_copy(b_hbm, b_v)
    cp_stage.wait()
    plsc.subcore_barrier()
    pltpu.sync_copy(table_sh, table_v)

    bias = b_v[...]
    zero = jnp.zeros((L,), jnp.float32)
    accs = (zero,) * CGRP
    for c in range(NCHUNK):
      buf = bufs[c % NB]
      cps[c].wait()

      def row(i, a, buf=buf):
        return tuple(
            a[g] + plsc.load_gather(table_v, [buf[i, pl.ds(g * L, L)]])
            for g in range(CGRP)
        )

      accs = lax.fori_loop(0, RCHUNK, row, accs)
      if c + NB < NCHUNK:
        cps[c + NB] = pltpu.async_copy(
            text_hbm.at[pl.ds((c + NB) * RCHUNK, RCHUNK), pl.ds(base, COLS)],
            bufs[c % NB], sems[c % NB])

    for g in range(CGRP):
      out_v[pl.ds(g * L, L)] = accs[g] + bias
    pltpu.sync_copy(out_v, out_hbm.at[pl.ds(base, COLS)])

  return k


def kernel(text, w, b):
  w_flat = jnp.pad(w.reshape(VOCAB), (0, PAD_VOCAB - VOCAB))
  b16 = jnp.broadcast_to(b, (L,)).astype(jnp.float32)
  return _sc_kernel()(text, w_flat, b16)
